# Initial kernel scaffold; baseline (speedup 1.0000x reference)
import jax
import jax.numpy as jnp
from jax.experimental import pallas as pl


def kernel(x, edge_index, edge_attr, batch, params):
    raise NotImplementedError("write your pallas kernel here")


# speedup vs baseline: 1.0001x; 1.0001x over previous
import jax
import jax.numpy as jnp
from jax.experimental import pallas as pl


def _gin(h, src, dst, e_emb, W1, b1, W2, b2, edge_mask=None):
    msg = jax.nn.relu(h[src] + e_emb)
    if edge_mask is not None:
        msg = msg * edge_mask
    agg = jax.ops.segment_sum(msg, dst, num_segments=h.shape[0])
    out = h + agg
    out = jax.nn.relu(out @ W1 + b1) @ W2 + b2
    return jax.nn.relu(out)


def kernel(x, edge_index, edge_attr, batch, params):
    B = 128
    src = edge_index[0]
    dst = edge_index[1]
    h = x @ params['W_in']
    for l in range(2):
        h = _gin(h, src, dst, edge_attr @ params['front_We' + str(l)],
                 params['front_W1' + str(l)], params['front_b1' + str(l)],
                 params['front_W2' + str(l)], params['front_b2' + str(l)])
    x_encode = h
    hc = x @ params['Wc_in']
    for l in range(2):
        hc = _gin(hc, src, dst, edge_attr @ params['cau_We' + str(l)],
                  params['cau_W1' + str(l)], params['cau_b1' + str(l)],
                  params['cau_W2' + str(l)], params['cau_b2' + str(l)])
    node_cau = jax.nn.sigmoid(hc @ params['w_node'] + params['b_node'])
    edge_cau = jax.nn.sigmoid((hc[src] + hc[dst]) @ params['w_edge'] + params['b_edge'])
    hb = x_encode * node_cau
    for l in range(2):
        hb = _gin(hb, src, dst, edge_attr @ params['back_We' + str(l)],
                  params['back_W1' + str(l)], params['back_b1' + str(l)],
                  params['back_W2' + str(l)], params['back_b2' + str(l)],
                  edge_mask=edge_cau)
    sums = jax.ops.segment_sum(hb, batch, num_segments=B)
    cnt = jax.ops.segment_sum(jnp.ones((hb.shape[0], 1), jnp.float32), batch, num_segments=B)
    h_graph = sums / jnp.clip(cnt, 1.0)
    return h_graph @ params['Wp'] + params['bp']


# speedup vs baseline: 1.8807x; 1.8805x over previous
import functools

import jax
import jax.numpy as jnp
from jax import lax
from jax.experimental import pallas as pl
from jax.experimental.pallas import tpu as pltpu
from jax.experimental.pallas import tpu_sc as plsc

N = 10000
E = 320000
D = 128
DE = 16
B = 128
C = 10

RB = 128
NRB = 79
NPAD = NRB * RB
EC = 128
NEC = E // EC
NSUB = 16

_F32 = jnp.float32


def _inproj_body(x_ref, wa_ref, wb_ref, oa_ref, ob_ref):
    xb = x_ref[...]
    oa_ref[...] = jnp.dot(xb, wa_ref[...], preferred_element_type=_F32)
    ob_ref[...] = jnp.dot(xb, wb_ref[...], preferred_element_type=_F32)


def _tc_inproj(xp, wa, wb):
    w_spec = pl.BlockSpec((D, D), lambda i: (0, 0))
    return pl.pallas_call(
        _inproj_body,
        grid=(NRB,),
        in_specs=[pl.BlockSpec((RB, D), lambda i: (i, 0)), w_spec, w_spec],
        out_specs=[pl.BlockSpec((RB, D), lambda i: (i, 0))] * 2,
        out_shape=[jax.ShapeDtypeStruct((NPAD, D), _F32)] * 2,
    )(xp, wa, wb)


_EB = 3200
_NEB = E // _EB


def _eemb_body(ea_ref, *refs):
    ws = refs[:6]
    outs = refs[6:]
    eb = ea_ref[...]
    for w_ref, o_ref in zip(ws, outs):
        o_ref[...] = jnp.dot(eb, w_ref[...], preferred_element_type=_F32)


def _tc_eemb(ea, ws):
    w_spec = pl.BlockSpec((DE, D), lambda i: (0, 0))
    return pl.pallas_call(
        _eemb_body,
        grid=(_NEB,),
        in_specs=[pl.BlockSpec((_EB, DE), lambda i: (i, 0))] + [w_spec] * 6,
        out_specs=[pl.BlockSpec((_EB, D), lambda i: (i, 0))] * 6,
        out_shape=[jax.ShapeDtypeStruct((E, D), _F32)] * 6,
    )(ea, *ws)


def _gin_mlp(h, agg, w1_ref, b1_ref, w2_ref, b2_ref):
    t = h + agg
    t = jnp.maximum(
        jnp.dot(t, w1_ref[...], preferred_element_type=_F32) + b1_ref[...], 0.0)
    t = jnp.dot(t, w2_ref[...], preferred_element_type=_F32) + b2_ref[...]
    return jnp.maximum(t, 0.0)


def _mlp2_body(ha_ref, aa_ref, hb_ref, ab_ref,
               w1a, b1a, w2a, b2a, w1b, b1b, w2b, b2b,
               oa_ref, ob_ref):
    oa_ref[...] = _gin_mlp(ha_ref[...], aa_ref[...], w1a, b1a, w2a, b2a)
    ob_ref[...] = _gin_mlp(hb_ref[...], ab_ref[...], w1b, b1b, w2b, b2b)


def _tc_mlp2(ha, aa, hb, ab, wts_a, wts_b):
    w_spec = pl.BlockSpec((D, D), lambda i: (0, 0))
    b_spec = pl.BlockSpec((1, D), lambda i: (0, 0))
    blk = pl.BlockSpec((RB, D), lambda i: (i, 0))
    return pl.pallas_call(
        _mlp2_body,
        grid=(NRB,),
        in_specs=[blk] * 4 + [w_spec, b_spec, w_spec, b_spec] * 2,
        out_specs=[blk] * 2,
        out_shape=[jax.ShapeDtypeStruct((NPAD, D), _F32)] * 2,
    )(ha, aa, hb, ab, *wts_a, *wts_b)


def _mlp2post_body(ha_ref, aa_ref, hb_ref, ab_ref,
                   w1a, b1a, w2a, b2a, w1b, b1b, w2b, b2b,
                   wn_ref, bn_ref, we_ref,
                   hb0_ref, s16_ref):
    x_enc = _gin_mlp(ha_ref[...], aa_ref[...], w1a, b1a, w2a, b2a)
    hc2 = _gin_mlp(hb_ref[...], ab_ref[...], w1b, b1b, w2b, b2b)
    s_n = jnp.dot(hc2, wn_ref[...], preferred_element_type=_F32) + bn_ref[...]
    ncau = jax.nn.sigmoid(s_n[:, 0:1])
    hb0_ref[...] = x_enc * ncau
    s_e = jnp.dot(hc2, we_ref[...], preferred_element_type=_F32)
    s16_ref[...] = jnp.broadcast_to(s_e[:, 0:1], (RB, DE))


def _tc_mlp2post(ha, aa, hb, ab, wts_a, wts_b, wn_pad, bn_row, we_pad):
    w_spec = pl.BlockSpec((D, D), lambda i: (0, 0))
    b_spec = pl.BlockSpec((1, D), lambda i: (0, 0))
    blk = pl.BlockSpec((RB, D), lambda i: (i, 0))
    return pl.pallas_call(
        _mlp2post_body,
        grid=(NRB,),
        in_specs=[blk] * 4 + [w_spec, b_spec, w_spec, b_spec] * 2
                 + [w_spec, b_spec, w_spec],
        out_specs=[blk, pl.BlockSpec((RB, DE), lambda i: (i, 0))],
        out_shape=[jax.ShapeDtypeStruct((NPAD, D), _F32),
                   jax.ShapeDtypeStruct((NPAD, DE), _F32)],
    )(ha, aa, hb, ab, *wts_a, *wts_b, wn_pad, bn_row, we_pad)


def _mlp1_body(h_ref, a2_ref, w1, b1, w2, b2, o_ref):
    agg = a2_ref[0] + a2_ref[1]
    o_ref[...] = _gin_mlp(h_ref[...], agg, w1, b1, w2, b2)


def _tc_mlp1(h, agg2, wts):
    w_spec = pl.BlockSpec((D, D), lambda i: (0, 0))
    b_spec = pl.BlockSpec((1, D), lambda i: (0, 0))
    blk = pl.BlockSpec((RB, D), lambda i: (i, 0))
    return pl.pallas_call(
        _mlp1_body,
        grid=(NRB,),
        in_specs=[blk, pl.BlockSpec((2, RB, D), lambda i: (0, i, 0)),
                  w_spec, b_spec, w_spec, b_spec],
        out_specs=blk,
        out_shape=jax.ShapeDtypeStruct((NPAD, D), _F32),
    )(h, agg2, *wts)


def _pool_body(hb_ref, b_ref, wp_ref, bp_ref, o_ref, sums, cnt):
    i = pl.program_id(0)

    @pl.when(i == 0)
    def _():
        sums[...] = jnp.zeros((B, D), _F32)
        cnt[...] = jnp.zeros((B, D), _F32)

    brow = b_ref[0, 0, :]
    iota_g = lax.broadcasted_iota(jnp.int32, (B, RB), 0)
    m = (brow[None, :] == iota_g).astype(_F32)
    sums[...] += jnp.dot(m, hb_ref[...], preferred_element_type=_F32)
    cnt[...] += jnp.broadcast_to(jnp.sum(m, axis=1, keepdims=True), (B, D))

    @pl.when(i == NRB - 1)
    def _():
        hg = sums[...] / jnp.maximum(cnt[...], 1.0)
        o_ref[...] = jnp.dot(hg, wp_ref[...], preferred_element_type=_F32) \
            + bp_ref[...]


def _tc_pool(hb, batchp, wp_pad, bp_row):
    return pl.pallas_call(
        _pool_body,
        grid=(NRB,),
        in_specs=[pl.BlockSpec((RB, D), lambda i: (i, 0)),
                  pl.BlockSpec((1, 1, RB), lambda i: (i, 0, 0)),
                  pl.BlockSpec((D, D), lambda i: (0, 0)),
                  pl.BlockSpec((1, D), lambda i: (0, 0))],
        out_specs=pl.BlockSpec((B, D), lambda i: (0, 0)),
        out_shape=jax.ShapeDtypeStruct((B, D), _F32),
        scratch_shapes=[pltpu.VMEM((B, D), _F32), pltpu.VMEM((B, D), _F32)],
    )(hb, batchp, wp_pad, bp_row)


_SC_MESH = plsc.VectorSubcoreMesh(core_axis_name="c", subcore_axis_name="s")


def _zero_tile_buf(buf):
    def zb(e, _):
        for v in range(D // 16):
            buf[e, pl.ds(v * 16, 16)] = jnp.zeros((16,), _F32)
        return 0
    lax.fori_loop(0, EC, zb, 0)


def _zero_agg(sid, zeros_buf, agg):
    nz = (NRB + NSUB - 1 - sid) // NSUB
    def zc(i, _):
        r = (sid + i * NSUB) * RB
        pltpu.sync_copy(zeros_buf, agg.at[pl.ds(r, RB)])
        return 0
    lax.fori_loop(0, nz, zc, 0)
    return nz


def _edge_chunk_loop(sid, lo, hi, src_hbm, dst_hbm, e_hbm, h_hbm,
                     sidx, didx, rows, emb, agg, sem,
                     ecau_hbm=None, ecv=None):
    nc = (hi - lo + NSUB - 1 - sid) // NSUB

    def body(i, _):
        base = (lo + sid + i * NSUB) * EC
        pltpu.sync_copy(src_hbm.at[pl.ds(base, EC)], sidx)
        pltpu.sync_copy(dst_hbm.at[pl.ds(base, EC)], didx)
        pltpu.sync_copy(e_hbm.at[pl.ds(base, EC)], emb)
        if ecau_hbm is not None:
            pltpu.sync_copy(ecau_hbm.at[pl.ds(base, EC)], ecv)
        pltpu.async_copy(h_hbm.at[sidx], rows, sem).wait()

        if ecau_hbm is None:
            def eb(e, _):
                for v in range(D // 16):
                    sl = pl.ds(v * 16, 16)
                    emb[e, sl] = jnp.maximum(rows[e, sl] + emb[e, sl], 0.0)
                return 0
        else:
            def eb(e, _):
                e16 = jnp.broadcast_to(e, (16,)).astype(jnp.int32)
                cau = plsc.load_gather(ecv, [e16])
                for v in range(D // 16):
                    sl = pl.ds(v * 16, 16)
                    emb[e, sl] = jnp.maximum(
                        rows[e, sl] + emb[e, sl], 0.0) * cau
                return 0
        lax.fori_loop(0, EC, eb, 0)
        pltpu.sync_copy(emb, agg.at[didx], add=True)
        return 0

    lax.fori_loop(0, nc, body, 0)


def _sc_edge_dual(ha, hb, ea, eb, src, dst):
    @functools.partial(
        pl.kernel, mesh=_SC_MESH,
        compiler_params=pltpu.CompilerParams(needs_layout_passes=False),
        out_type=(jax.ShapeDtypeStruct((NPAD, D), _F32),) * 2,
        scratch_types=(
            pltpu.VMEM((EC,), jnp.int32),
            pltpu.VMEM((EC,), jnp.int32),
            pltpu.VMEM((EC, D), _F32),
            pltpu.VMEM((EC, D), _F32),
            pltpu.VMEM_SHARED((NPAD, D), _F32),
            pltpu.SemaphoreType.DMA,
        ))
    def k(ha_hbm, hb_hbm, ea_hbm, eb_hbm, src_hbm, dst_hbm,
          outa, outb, sidx, didx, rows, emb, agg, sem):
        cid = lax.axis_index("c")
        sid = lax.axis_index("s")
        _zero_tile_buf(rows)
        _zero_agg(sid, rows, agg)
        plsc.subcore_barrier()

        @pl.when(cid == 0)
        def _():
            _edge_chunk_loop(sid, 0, NEC, src_hbm, dst_hbm, ea_hbm, ha_hbm,
                             sidx, didx, rows, emb, agg, sem)

        @pl.when(cid == 1)
        def _():
            _edge_chunk_loop(sid, 0, NEC, src_hbm, dst_hbm, eb_hbm, hb_hbm,
                             sidx, didx, rows, emb, agg, sem)

        plsc.subcore_barrier()
        nz = (NRB + NSUB - 1 - sid) // NSUB

        def wc(i, _):
            r = (sid + i * NSUB) * RB

            @pl.when(cid == 0)
            def _():
                pltpu.sync_copy(agg.at[pl.ds(r, RB)], outa.at[pl.ds(r, RB)])

            @pl.when(cid == 1)
            def _():
                pltpu.sync_copy(agg.at[pl.ds(r, RB)], outb.at[pl.ds(r, RB)])
            return 0

        lax.fori_loop(0, nz, wc, 0)

    return k(ha, hb, ea, eb, src, dst)


def _sc_edge_masked(h, e, src, dst, ecau):
    half = NEC // 2

    @functools.partial(
        pl.kernel, mesh=_SC_MESH,
        compiler_params=pltpu.CompilerParams(needs_layout_passes=False),
        out_type=jax.ShapeDtypeStruct((2, NPAD, D), _F32),
        scratch_types=(
            pltpu.VMEM((EC,), jnp.int32),
            pltpu.VMEM((EC,), jnp.int32),
            pltpu.VMEM((EC, D), _F32),
            pltpu.VMEM((EC, D), _F32),
            pltpu.VMEM((EC,), _F32),
            pltpu.VMEM_SHARED((NPAD, D), _F32),
            pltpu.SemaphoreType.DMA,
        ))
    def k(h_hbm, e_hbm, src_hbm, dst_hbm, ecau_hbm,
          outp, sidx, didx, rows, emb, ecv, agg, sem):
        cid = lax.axis_index("c")
        sid = lax.axis_index("s")
        _zero_tile_buf(rows)
        _zero_agg(sid, rows, agg)
        plsc.subcore_barrier()

        @pl.when(cid == 0)
        def _():
            _edge_chunk_loop(sid, 0, half, src_hbm, dst_hbm, e_hbm, h_hbm,
                             sidx, didx, rows, emb, agg, sem,
                             ecau_hbm=ecau_hbm, ecv=ecv)

        @pl.when(cid == 1)
        def _():
            _edge_chunk_loop(sid, half, NEC, src_hbm, dst_hbm, e_hbm, h_hbm,
                             sidx, didx, rows, emb, agg, sem,
                             ecau_hbm=ecau_hbm, ecv=ecv)

        plsc.subcore_barrier()
        nz = (NRB + NSUB - 1 - sid) // NSUB

        def wc(i, _):
            r = (sid + i * NSUB) * RB
            pltpu.sync_copy(agg.at[pl.ds(r, RB)], outp.at[cid, pl.ds(r, RB)])
            return 0

        lax.fori_loop(0, nz, wc, 0)

    return k(h, e, src, dst, ecau)


def _sc_ecau(s1, src, dst, be16):
    NW = 32

    @functools.partial(
        pl.kernel, mesh=_SC_MESH,
        compiler_params=pltpu.CompilerParams(needs_layout_passes=False),
        out_type=jax.ShapeDtypeStruct((E,), _F32),
        scratch_types=(
            pltpu.VMEM((EC,), jnp.int32),
            pltpu.VMEM((EC,), jnp.int32),
            pltpu.VMEM((EC,), _F32),
            pltpu.VMEM((NPAD,), _F32),
            pltpu.VMEM((16,), _F32),
        ))
    def k(s_hbm, src_hbm, dst_hbm, be_hbm,
          out, sidx, didx, ocv, sv_t, bev):
        cid = lax.axis_index("c")
        sid = lax.axis_index("s")
        wid = sid * 2 + cid
        pltpu.sync_copy(be_hbm, bev)
        pltpu.sync_copy(s_hbm, sv_t)
        nc = (NEC + NW - 1 - wid) // NW

        def body(i, _):
            base = (wid + i * NW) * EC
            pltpu.sync_copy(src_hbm.at[pl.ds(base, EC)], sidx)
            pltpu.sync_copy(dst_hbm.at[pl.ds(base, EC)], didx)
            bv = bev[...]
            for g in range(EC // 16):
                sl = pl.ds(g * 16, 16)
                sv = plsc.load_gather(sv_t, [sidx[sl]])
                dv = plsc.load_gather(sv_t, [didx[sl]])
                t = sv + dv + bv
                ocv[sl] = 1.0 / (1.0 + jnp.exp(-t))
            pltpu.sync_copy(ocv, out.at[pl.ds(base, EC)])
            return 0

        lax.fori_loop(0, nc, body, 0)

    return k(s1, src, dst, be16)


def _wts(p, name, l):
    return (p[name + '_W1' + str(l)], p[name + '_b1' + str(l)].reshape(1, D),
            p[name + '_W2' + str(l)], p[name + '_b2' + str(l)].reshape(1, D))


def kernel(x, edge_index, edge_attr, batch, params):
    p = params
    src = edge_index[0].astype(jnp.int32)
    dst = edge_index[1].astype(jnp.int32)
    xp = jnp.pad(x, ((0, NPAD - N), (0, 0)))
    batchp = jnp.pad(batch.astype(jnp.int32), (0, NPAD - N),
                     constant_values=B).reshape(NRB, 1, RB)

    h, hc = _tc_inproj(xp, p['W_in'], p['Wc_in'])
    ef0, ec0, eb0, ef1, ec1, eb1 = _tc_eemb(
        edge_attr, [p['front_We0'], p['cau_We0'], p['back_We0'],
                    p['front_We1'], p['cau_We1'], p['back_We1']])

    agg_f, agg_c = _sc_edge_dual(h, hc, ef0, ec0, src, dst)
    h, hc = _tc_mlp2(h, agg_f, hc, agg_c, _wts(p, 'front', 0), _wts(p, 'cau', 0))

    agg_f, agg_c = _sc_edge_dual(h, hc, ef1, ec1, src, dst)
    wn_pad = jnp.pad(p['w_node'], ((0, 0), (0, D - 1)))
    we_pad = jnp.pad(p['w_edge'], ((0, 0), (0, D - 1)))
    bn_row = jnp.broadcast_to(p['b_node'], (1, D))
    hb, s16 = _tc_mlp2post(h, agg_f, hc, agg_c,
                           _wts(p, 'front', 1), _wts(p, 'cau', 1),
                           wn_pad, bn_row, we_pad)

    be16 = jnp.broadcast_to(p['b_edge'], (16,)).astype(_F32)
    ecau = _sc_ecau(s16[:, 0], src, dst, be16)

    for l, e_emb in ((0, eb0), (1, eb1)):
        agg2 = _sc_edge_masked(hb, e_emb, src, dst, ecau)
        hb = _tc_mlp1(hb, agg2, _wts(p, 'back', l))

    wp_pad = jnp.pad(p['Wp'], ((0, 0), (0, D - C)))
    bp_row = jnp.pad(p['bp'], (0, D - C)).reshape(1, D)
    out = _tc_pool(hb, batchp, wp_pad, bp_row)
    return out[:, :C]
